# baseline (speedup 1.0000x reference)
import functools

import jax
import jax.numpy as jnp
from jax import lax
from jax.experimental import pallas as pl
from jax.experimental.pallas import tpu as pltpu
from jax.experimental.pallas import tpu_sc as plsc

VOCAB = 100000
HIDDEN = 128
MAX_POS = 512
B, L = 4096, 50
N = B * L
S = 8
SPLIT_B = B // S
NH = SPLIT_B * L
NC, NS = 2, 16
NW = NC * NS
PER_W = NH // NW
C = 80
CHUNKS = PER_W // C
LANES = 8
U = 4
TB = 128
EPS = 1e-12


def _sc_gather_add(word_table, pos_table, ids, pids):
    mesh = plsc.VectorSubcoreMesh(core_axis_name="c", subcore_axis_name="s")

    @functools.partial(
        pl.kernel,
        mesh=mesh,
        out_type=jax.ShapeDtypeStruct((NH, HIDDEN), jnp.float32),
        scratch_types=[
            pltpu.VMEM((C,), jnp.int32),
            pltpu.VMEM((C,), jnp.int32),
            pltpu.VMEM((C,), jnp.int32),
            pltpu.VMEM((C,), jnp.int32),
            pltpu.VMEM((C, HIDDEN), jnp.float32),
            pltpu.VMEM((C, HIDDEN), jnp.float32),
            pltpu.VMEM((C, HIDDEN), jnp.float32),
            pltpu.VMEM((C, HIDDEN), jnp.float32),
            pltpu.VMEM((C, HIDDEN), jnp.float32),
            pltpu.VMEM((C, HIDDEN), jnp.float32),
            pltpu.SemaphoreType.DMA,
            pltpu.SemaphoreType.DMA,
            pltpu.SemaphoreType.DMA,
            pltpu.SemaphoreType.DMA,
            pltpu.SemaphoreType.DMA,
            pltpu.SemaphoreType.DMA,
        ],
    )
    def k(word_hbm, pos_hbm, ids_hbm, pids_hbm, out_hbm,
          ids0, ids1, pids0, pids1, rows0, rows1, prows0, prows1, ob0, ob1,
          isem0, isem1, gsem0, gsem1, wsem0, wsem1):
        wid = lax.axis_index("s") * NC + lax.axis_index("c")
        base = wid * PER_W

        def compute(rows_b, prows_b, ob_b):
            def tok_body(ti, tc):
                for u in range(U):
                    t = ti * U + u
                    for j in range(LANES):
                        ob_b[t, pl.ds(16 * j, 16)] = \
                            rows_b[t, pl.ds(16 * j, 16)] + \
                            prows_b[t, pl.ds(16 * j, 16)]
                return tc
            lax.fori_loop(0, C // U, tok_body, 0)

        def chunk_step(ci, bufs):
            ids_b, pids_b, rows_b, prows_b, ob_b, isem, gsem, wsem = bufs
            off = base + ci * C
            pltpu.make_async_copy(word_hbm.at[ids_b], rows_b, gsem).wait()
            pltpu.make_async_copy(pos_hbm.at[pids_b], prows_b, gsem).wait()

            @pl.when(ci >= 2)
            def _():
                pltpu.make_async_copy(
                    ob_b, out_hbm.at[pl.ds(off - 2 * C, C)], wsem).wait()

            @pl.when(ci + 2 < CHUNKS)
            def _():
                off2 = off + 2 * C
                pltpu.make_async_copy(
                    ids_hbm.at[pl.ds(off2, C)], ids_b, isem).start()
                pltpu.make_async_copy(
                    pids_hbm.at[pl.ds(off2, C)], pids_b, isem).start()

            compute(rows_b, prows_b, ob_b)
            pltpu.make_async_copy(ob_b, out_hbm.at[pl.ds(off, C)], wsem).start()

            @pl.when(ci + 2 < CHUNKS)
            def _():
                off2 = off + 2 * C
                pltpu.make_async_copy(
                    ids_hbm.at[pl.ds(off2, C)], ids_b, isem).wait()
                pltpu.make_async_copy(
                    pids_hbm.at[pl.ds(off2, C)], pids_b, isem).wait()
                pltpu.make_async_copy(word_hbm.at[ids_b], rows_b, gsem).start()
                pltpu.make_async_copy(pos_hbm.at[pids_b], prows_b, gsem).start()

        slot0 = (ids0, pids0, rows0, prows0, ob0, isem0, gsem0, wsem0)
        slot1 = (ids1, pids1, rows1, prows1, ob1, isem1, gsem1, wsem1)

        pltpu.sync_copy(ids_hbm.at[pl.ds(base, C)], ids0)
        pltpu.sync_copy(pids_hbm.at[pl.ds(base, C)], pids0)
        pltpu.sync_copy(ids_hbm.at[pl.ds(base + C, C)], ids1)
        pltpu.sync_copy(pids_hbm.at[pl.ds(base + C, C)], pids1)
        pltpu.make_async_copy(word_hbm.at[ids0], rows0, gsem0).start()
        pltpu.make_async_copy(pos_hbm.at[pids0], prows0, gsem0).start()
        pltpu.make_async_copy(word_hbm.at[ids1], rows1, gsem1).start()
        pltpu.make_async_copy(pos_hbm.at[pids1], prows1, gsem1).start()

        def pair_body(p, carry):
            chunk_step(2 * p, slot0)
            chunk_step(2 * p + 1, slot1)
            return carry

        lax.fori_loop(0, CHUNKS // 2, pair_body, 0)

        endo = base + (CHUNKS - 2) * C
        pltpu.make_async_copy(ob0, out_hbm.at[pl.ds(endo, C)], wsem0).wait()
        pltpu.make_async_copy(ob1, out_hbm.at[pl.ds(endo + C, C)], wsem1).wait()

    return k(word_table, pos_table, ids, pids)


def _tc_layernorm_split(x, gamma, beta, split, prev):
    hb = split * (SPLIT_B // TB)

    def body(x_ref, g_ref, b_ref, *rest):
        o_ref = rest[-1]
        xv = x_ref[...]
        mean = jnp.mean(xv, axis=1, keepdims=True)
        xc = xv - mean
        var = jnp.mean(xc * xc, axis=1, keepdims=True)
        y = xc * lax.rsqrt(var + EPS) * g_ref[...] + b_ref[...]
        for bb in range(TB):
            o_ref[bb, :, :] = y[bb * L:(bb + 1) * L, :]

    in_specs = [
        pl.BlockSpec((TB * L, HIDDEN), lambda i: (i, 0)),
        pl.BlockSpec((1, HIDDEN), lambda i: (0, 0)),
        pl.BlockSpec((1, HIDDEN), lambda i: (0, 0)),
    ]
    args = [x, gamma.reshape(1, HIDDEN), beta.reshape(1, HIDDEN)]
    aliases = {}
    if prev is not None:
        in_specs.append(pl.BlockSpec(memory_space=pl.ANY))
        args.append(prev)
        aliases = {3: 0}

    return pl.pallas_call(
        body,
        grid=(SPLIT_B // TB,),
        in_specs=in_specs,
        out_specs=pl.BlockSpec((TB, L, HIDDEN), lambda i: (hb + i, 0, 0)),
        out_shape=jax.ShapeDtypeStruct((B, L, HIDDEN), jnp.float32),
        input_output_aliases=aliases,
    )(*args)


def kernel(input_ids, position_ids, text_attn_masks, word_table, pos_table,
           ln_gamma, ln_beta):
    ids = input_ids.reshape(-1).astype(jnp.int32)
    pids = position_ids.reshape(-1).astype(jnp.int32)
    xs = [_sc_gather_add(word_table, pos_table,
                         ids[s * NH:(s + 1) * NH], pids[s * NH:(s + 1) * NH])
          for s in range(S)]
    out = None
    for s in range(S):
        out = _tc_layernorm_split(xs[s], ln_gamma, ln_beta, s, out)
    return (out, text_attn_masks)


# speedup vs baseline: 1.0509x; 1.0509x over previous
import functools

import jax
import jax.numpy as jnp
from jax import lax
from jax.experimental import pallas as pl
from jax.experimental.pallas import tpu as pltpu
from jax.experimental.pallas import tpu_sc as plsc

VOCAB = 100000
HIDDEN = 128
MAX_POS = 512
B, L = 4096, 50
N = B * L
S = 4
SPLIT_B = B // S
NH = SPLIT_B * L
NC, NS = 2, 16
NW = NC * NS
PER_W = NH // NW
C = 80
CHUNKS = PER_W // C
LANES = 8
U = 4
TB = 128
EPS = 1e-12


def _sc_gather_add(word_table, pos_table, ids, pids):
    mesh = plsc.VectorSubcoreMesh(core_axis_name="c", subcore_axis_name="s")

    @functools.partial(
        pl.kernel,
        mesh=mesh,
        out_type=jax.ShapeDtypeStruct((NH, HIDDEN), jnp.float32),
        scratch_types=[
            pltpu.VMEM((C,), jnp.int32),
            pltpu.VMEM((C,), jnp.int32),
            pltpu.VMEM((C,), jnp.int32),
            pltpu.VMEM((C,), jnp.int32),
            pltpu.VMEM((C, HIDDEN), jnp.float32),
            pltpu.VMEM((C, HIDDEN), jnp.float32),
            pltpu.VMEM((C, HIDDEN), jnp.float32),
            pltpu.VMEM((C, HIDDEN), jnp.float32),
            pltpu.VMEM((C, HIDDEN), jnp.float32),
            pltpu.VMEM((C, HIDDEN), jnp.float32),
            pltpu.SemaphoreType.DMA,
            pltpu.SemaphoreType.DMA,
            pltpu.SemaphoreType.DMA,
            pltpu.SemaphoreType.DMA,
            pltpu.SemaphoreType.DMA,
            pltpu.SemaphoreType.DMA,
        ],
    )
    def k(word_hbm, pos_hbm, ids_hbm, pids_hbm, out_hbm,
          ids0, ids1, pids0, pids1, rows0, rows1, prows0, prows1, ob0, ob1,
          isem0, isem1, gsem0, gsem1, wsem0, wsem1):
        wid = lax.axis_index("s") * NC + lax.axis_index("c")
        base = wid * PER_W

        def compute(rows_b, prows_b, ob_b):
            def tok_body(ti, tc):
                for u in range(U):
                    t = ti * U + u
                    for j in range(LANES):
                        ob_b[t, pl.ds(16 * j, 16)] = \
                            rows_b[t, pl.ds(16 * j, 16)] + \
                            prows_b[t, pl.ds(16 * j, 16)]
                return tc
            lax.fori_loop(0, C // U, tok_body, 0)

        def chunk_step(ci, bufs):
            ids_b, pids_b, rows_b, prows_b, ob_b, isem, gsem, wsem = bufs
            off = base + ci * C
            pltpu.make_async_copy(word_hbm.at[ids_b], rows_b, gsem).wait()
            pltpu.make_async_copy(pos_hbm.at[pids_b], prows_b, gsem).wait()

            @pl.when(ci >= 2)
            def _():
                pltpu.make_async_copy(
                    ob_b, out_hbm.at[pl.ds(off - 2 * C, C)], wsem).wait()

            @pl.when(ci + 2 < CHUNKS)
            def _():
                off2 = off + 2 * C
                pltpu.make_async_copy(
                    ids_hbm.at[pl.ds(off2, C)], ids_b, isem).start()
                pltpu.make_async_copy(
                    pids_hbm.at[pl.ds(off2, C)], pids_b, isem).start()

            compute(rows_b, prows_b, ob_b)
            pltpu.make_async_copy(ob_b, out_hbm.at[pl.ds(off, C)], wsem).start()

            @pl.when(ci + 2 < CHUNKS)
            def _():
                off2 = off + 2 * C
                pltpu.make_async_copy(
                    ids_hbm.at[pl.ds(off2, C)], ids_b, isem).wait()
                pltpu.make_async_copy(
                    pids_hbm.at[pl.ds(off2, C)], pids_b, isem).wait()
                pltpu.make_async_copy(word_hbm.at[ids_b], rows_b, gsem).start()
                pltpu.make_async_copy(pos_hbm.at[pids_b], prows_b, gsem).start()

        slot0 = (ids0, pids0, rows0, prows0, ob0, isem0, gsem0, wsem0)
        slot1 = (ids1, pids1, rows1, prows1, ob1, isem1, gsem1, wsem1)

        pltpu.sync_copy(ids_hbm.at[pl.ds(base, C)], ids0)
        pltpu.sync_copy(pids_hbm.at[pl.ds(base, C)], pids0)
        pltpu.sync_copy(ids_hbm.at[pl.ds(base + C, C)], ids1)
        pltpu.sync_copy(pids_hbm.at[pl.ds(base + C, C)], pids1)
        pltpu.make_async_copy(word_hbm.at[ids0], rows0, gsem0).start()
        pltpu.make_async_copy(pos_hbm.at[pids0], prows0, gsem0).start()
        pltpu.make_async_copy(word_hbm.at[ids1], rows1, gsem1).start()
        pltpu.make_async_copy(pos_hbm.at[pids1], prows1, gsem1).start()

        def pair_body(p, carry):
            chunk_step(2 * p, slot0)
            chunk_step(2 * p + 1, slot1)
            return carry

        lax.fori_loop(0, CHUNKS // 2, pair_body, 0)

        endo = base + (CHUNKS - 2) * C
        pltpu.make_async_copy(ob0, out_hbm.at[pl.ds(endo, C)], wsem0).wait()
        pltpu.make_async_copy(ob1, out_hbm.at[pl.ds(endo + C, C)], wsem1).wait()

    return k(word_table, pos_table, ids, pids)


def _tc_layernorm_split(x, gamma, beta, split, prev):
    hb = split * (SPLIT_B // TB)

    def body(x_ref, g_ref, b_ref, *rest):
        o_ref = rest[-1]
        xv = x_ref[...]
        mean = jnp.mean(xv, axis=1, keepdims=True)
        xc = xv - mean
        var = jnp.mean(xc * xc, axis=1, keepdims=True)
        y = xc * lax.rsqrt(var + EPS) * g_ref[...] + b_ref[...]
        for bb in range(TB):
            o_ref[bb, :, :] = y[bb * L:(bb + 1) * L, :]

    in_specs = [
        pl.BlockSpec((TB * L, HIDDEN), lambda i: (i, 0)),
        pl.BlockSpec((1, HIDDEN), lambda i: (0, 0)),
        pl.BlockSpec((1, HIDDEN), lambda i: (0, 0)),
    ]
    args = [x, gamma.reshape(1, HIDDEN), beta.reshape(1, HIDDEN)]
    aliases = {}
    if prev is not None:
        in_specs.append(pl.BlockSpec(memory_space=pl.ANY))
        args.append(prev)
        aliases = {3: 0}

    return pl.pallas_call(
        body,
        grid=(SPLIT_B // TB,),
        in_specs=in_specs,
        out_specs=pl.BlockSpec((TB, L, HIDDEN), lambda i: (hb + i, 0, 0)),
        out_shape=jax.ShapeDtypeStruct((B, L, HIDDEN), jnp.float32),
        input_output_aliases=aliases,
    )(*args)


def kernel(input_ids, position_ids, text_attn_masks, word_table, pos_table,
           ln_gamma, ln_beta):
    ids = input_ids.reshape(-1).astype(jnp.int32)
    pids = position_ids.reshape(-1).astype(jnp.int32)
    xs = [_sc_gather_add(word_table, pos_table,
                         ids[s * NH:(s + 1) * NH], pids[s * NH:(s + 1) * NH])
          for s in range(S)]
    out = None
    for s in range(S):
        out = _tc_layernorm_split(xs[s], ln_gamma, ln_beta, s, out)
    return (out, text_attn_masks)
